# baseline (speedup 1.0000x reference)
import jax
import jax.numpy as jnp
from jax import lax
from jax.experimental import pallas as pl
from jax.experimental.pallas import tpu as pltpu
from jax.experimental.pallas import tpu_sc as plsc

D = 1024
F = 2048
E = 8
T = 2048
A = 2 * T
BLK = 128
NBLK = A // BLK + E
XS = NBLK * BLK
NC, NS = 2, 16
NW = NC * NS
TPW = T // NW

_PREC = lax.Precision.DEFAULT


def _router_body(h_ref, wr_ref, dest_ref, w_ref, nblk_ref):
    h = h_ref[...]
    logits = jnp.dot(h, wr_ref[...], preferred_element_type=jnp.float32,
                     precision=_PREC)
    iota = lax.broadcasted_iota(jnp.int32, (T, E), 1)
    m0 = jnp.max(logits, axis=1, keepdims=True)
    e0 = jnp.min(jnp.where(logits == m0, iota, E), axis=1, keepdims=True)
    masked = jnp.where(iota == e0, -jnp.inf, logits)
    m1 = jnp.max(masked, axis=1, keepdims=True)
    e1 = jnp.min(jnp.where(masked == m1, iota, E), axis=1, keepdims=True)
    t = jnp.exp(m1 - m0)
    w0 = 1.0 / (1.0 + t)
    w1 = t * w0

    oh0 = (iota == e0).astype(jnp.int32)
    oh1 = (iota == e1).astype(jnp.int32)
    occ = jnp.concatenate([oh0, oh1], axis=0)
    row = lax.broadcasted_iota(jnp.int32, (A, E), 0)
    s = occ
    d = 1
    while d < A:
        s = s + jnp.where(row >= d, pltpu.roll(s, d, 0), 0)
        d *= 2
    excl = s - occ
    counts = s[A - 1:A, :]
    nblk = (counts + (BLK - 1)) >> 7
    lane = lax.broadcasted_iota(jnp.int32, (1, E), 1)
    c = nblk
    d = 1
    while d < E:
        c = c + jnp.where(lane >= d, pltpu.roll(c, d, 1), 0)
        d *= 2
    pad_start = (c - nblk) << 7

    rank0 = jnp.sum(excl[:T] * oh0, axis=1, keepdims=True)
    rank1 = jnp.sum(excl[T:] * oh1, axis=1, keepdims=True)
    ps0 = jnp.sum(pad_start * oh0, axis=1, keepdims=True)
    ps1 = jnp.sum(pad_start * oh1, axis=1, keepdims=True)
    dest_ref[:, 0:1] = ps0 + rank0
    dest_ref[:, 1:2] = ps1 + rank1
    w_ref[:, 0:16] = jnp.broadcast_to(w0, (T, 16))
    w_ref[:, 16:32] = jnp.broadcast_to(w1, (T, 16))
    nblk_ref[...] = nblk


def _router(h, wr):
    return pl.pallas_call(
        _router_body,
        out_shape=(
            jax.ShapeDtypeStruct((T, 2), jnp.int32),
            jax.ShapeDtypeStruct((T, 32), jnp.float32),
            jax.ShapeDtypeStruct((1, E), jnp.int32),
        ),
    )(h, wr)


def _dispatch_body(h_hbm, d0_hbm, d1_hbm, xs_hbm, idx0_v, idx1_v, rows_v, sem):
    wid = lax.axis_index("s") * NC + lax.axis_index("c")
    base = wid * TPW
    pltpu.sync_copy(d0_hbm.at[pl.ds(base, TPW)], idx0_v)
    pltpu.sync_copy(d1_hbm.at[pl.ds(base, TPW)], idx1_v)
    pltpu.sync_copy(h_hbm.at[pl.ds(base, TPW)], rows_v)
    c0 = pltpu.async_copy(rows_v, xs_hbm.at[idx0_v], sem)
    c0.wait()
    c1 = pltpu.async_copy(rows_v, xs_hbm.at[idx1_v], sem)
    c1.wait()


def _dispatch(h, d0, d1):
    return pl.kernel(
        _dispatch_body,
        out_type=jax.ShapeDtypeStruct((XS, D), jnp.float32),
        mesh=plsc.VectorSubcoreMesh(core_axis_name="c", subcore_axis_name="s"),
        scratch_types=[
            pltpu.VMEM((TPW,), jnp.int32),
            pltpu.VMEM((TPW,), jnp.int32),
            pltpu.VMEM((TPW, D), jnp.float32),
            pltpu.SemaphoreType.DMA,
        ],
    )(h, d0, d1)


def _ffn_body(meta_ref, x_ref, wg_ref, wu_ref, wd_ref, o_ref):
    b = pl.program_id(0)

    @pl.when(b < meta_ref[0])
    def _():
        x = x_ref[...]
        g = jnp.dot(x, wg_ref[0], preferred_element_type=jnp.float32,
                    precision=_PREC)
        u = jnp.dot(x, wu_ref[0], preferred_element_type=jnp.float32,
                    precision=_PREC)
        act = (g * (1.0 / (1.0 + jnp.exp(-g)))) * u
        o_ref[...] = jnp.dot(act, wd_ref[0],
                             preferred_element_type=jnp.float32,
                             precision=_PREC)


def _ffn(meta, xs, wg, wu, wd):
    grid_spec = pltpu.PrefetchScalarGridSpec(
        num_scalar_prefetch=1,
        grid=(NBLK,),
        in_specs=[
            pl.BlockSpec((BLK, D), lambda b, meta: (b, 0)),
            pl.BlockSpec((1, D, F), lambda b, meta: (meta[1 + b], 0, 0)),
            pl.BlockSpec((1, D, F), lambda b, meta: (meta[1 + b], 0, 0)),
            pl.BlockSpec((1, F, D), lambda b, meta: (meta[1 + b], 0, 0)),
        ],
        out_specs=pl.BlockSpec((BLK, D), lambda b, meta: (b, 0)),
    )
    return pl.pallas_call(
        _ffn_body,
        grid_spec=grid_spec,
        out_shape=jax.ShapeDtypeStruct((XS, D), jnp.float32),
        compiler_params=pltpu.CompilerParams(
            dimension_semantics=("arbitrary",)),
    )(meta, xs, wg, wu, wd)


_CCH = 32


def _combine_body(ys_hbm, d0_hbm, d1_hbm, w_hbm, out_hbm,
                  idx0_v, idx1_v, w_v, ya_v, yb_v, sem):
    wid = lax.axis_index("s") * NC + lax.axis_index("c")
    for chunk in range(TPW // _CCH):
        base = wid * TPW + chunk * _CCH
        pltpu.sync_copy(d0_hbm.at[pl.ds(base, _CCH)], idx0_v)
        pltpu.sync_copy(d1_hbm.at[pl.ds(base, _CCH)], idx1_v)
        pltpu.sync_copy(w_hbm.at[pl.ds(base, _CCH)], w_v)
        ca = pltpu.async_copy(ys_hbm.at[idx0_v], ya_v, sem)
        cb = pltpu.async_copy(ys_hbm.at[idx1_v], yb_v, sem)
        ca.wait()
        cb.wait()

        def row_fn(r, _):
            w0v = w_v[r, pl.ds(0, 16)]
            w1v = w_v[r, pl.ds(16, 16)]
            for c in range(D // 16):
                a = ya_v[r, pl.ds(c * 16, 16)]
                b = yb_v[r, pl.ds(c * 16, 16)]
                ya_v[r, pl.ds(c * 16, 16)] = w0v * a + w1v * b
            return 0

        lax.fori_loop(0, _CCH, row_fn, 0)
        pltpu.sync_copy(ya_v, out_hbm.at[pl.ds(base, _CCH)])


def _combine(ys, d0, d1, w):
    return pl.kernel(
        _combine_body,
        out_type=jax.ShapeDtypeStruct((T, D), jnp.float32),
        mesh=plsc.VectorSubcoreMesh(core_axis_name="c", subcore_axis_name="s"),
        scratch_types=[
            pltpu.VMEM((_CCH,), jnp.int32),
            pltpu.VMEM((_CCH,), jnp.int32),
            pltpu.VMEM((_CCH, 32), jnp.float32),
            pltpu.VMEM((_CCH, D), jnp.float32),
            pltpu.VMEM((_CCH, D), jnp.float32),
            pltpu.SemaphoreType.DMA,
        ],
    )(ys, d0, d1, w)


def kernel(x, Wr, W_gate, W_up, W_down):
    B, S, _ = x.shape
    h = x.reshape(T, D)
    dest, w, nblk = _router(h, Wr)
    d0 = dest[:, 0]
    d1 = dest[:, 1]
    ends = jnp.cumsum(nblk[0])
    total = ends[E - 1]
    b_iota = jnp.arange(NBLK, dtype=jnp.int32)
    blk_e = jnp.sum((b_iota[:, None] >= ends[None, :]).astype(jnp.int32),
                    axis=1)
    e_iota = jnp.arange(E, dtype=jnp.int32)
    e_last = jnp.max(jnp.where(nblk[0] > 0, e_iota, 0))
    blk_e = jnp.minimum(blk_e, e_last)
    meta = jnp.concatenate([total[None], blk_e])
    xs = _dispatch(h, d0, d1)
    ys = _ffn(meta, xs, W_gate.astype(jnp.bfloat16),
              W_up.astype(jnp.bfloat16), W_down.astype(jnp.bfloat16))
    out = _combine(ys, d0, d1, w)
    return out.reshape(B, S, D)


# speedup vs baseline: 1.3064x; 1.3064x over previous
import jax
import jax.numpy as jnp
from jax import lax
from jax.experimental import pallas as pl
from jax.experimental.pallas import tpu as pltpu
from jax.experimental.pallas import tpu_sc as plsc

D = 1024
F = 2048
E = 8
T = 2048
A = 2 * T
BLK = 128
NBLK = A // BLK + E
XS = NBLK * BLK
NC, NS = 2, 16
NW = NC * NS
TPW = T // NW

_PREC = lax.Precision.DEFAULT


def _router_body(h_ref, wr_ref, d0_ref, d1_ref, w_ref, nblk_ref):
    h = h_ref[...]
    logits = jnp.dot(h, wr_ref[...], preferred_element_type=jnp.float32,
                     precision=_PREC)
    iota = lax.broadcasted_iota(jnp.int32, (T, E), 1)
    m0 = jnp.max(logits, axis=1, keepdims=True)
    e0 = jnp.min(jnp.where(logits == m0, iota, E), axis=1, keepdims=True)
    masked = jnp.where(iota == e0, -jnp.inf, logits)
    m1 = jnp.max(masked, axis=1, keepdims=True)
    e1 = jnp.min(jnp.where(masked == m1, iota, E), axis=1, keepdims=True)
    t = jnp.exp(m1 - m0)
    w0 = 1.0 / (1.0 + t)
    w1 = t * w0

    oh0 = (iota == e0).astype(jnp.int32)
    oh1 = (iota == e1).astype(jnp.int32)
    occ = jnp.concatenate([oh0, oh1], axis=0)
    row = lax.broadcasted_iota(jnp.int32, (A, E), 0)
    s = occ
    d = 1
    while d < A:
        s = s + jnp.where(row >= d, pltpu.roll(s, d, 0), 0)
        d *= 2
    excl = s - occ
    counts = s[A - 1:A, :]
    nblk = (counts + (BLK - 1)) >> 7
    lane = lax.broadcasted_iota(jnp.int32, (1, E), 1)
    c = nblk
    d = 1
    while d < E:
        c = c + jnp.where(lane >= d, pltpu.roll(c, d, 1), 0)
        d *= 2
    pad_start = (c - nblk) << 7

    rank0 = jnp.sum(excl[:T] * oh0, axis=1, keepdims=True)
    rank1 = jnp.sum(excl[T:] * oh1, axis=1, keepdims=True)
    ps0 = jnp.sum(pad_start * oh0, axis=1, keepdims=True)
    ps1 = jnp.sum(pad_start * oh1, axis=1, keepdims=True)
    d0_ref[...] = ps0 + rank0
    d1_ref[...] = ps1 + rank1
    w_ref[:, 0:16] = jnp.broadcast_to(w0, (T, 16))
    w_ref[:, 16:32] = jnp.broadcast_to(w1, (T, 16))
    nblk_ref[...] = nblk


def _router(h, wr):
    return pl.pallas_call(
        _router_body,
        out_shape=(
            jax.ShapeDtypeStruct((T, 1), jnp.int32),
            jax.ShapeDtypeStruct((T, 1), jnp.int32),
            jax.ShapeDtypeStruct((T, 32), jnp.float32),
            jax.ShapeDtypeStruct((1, E), jnp.int32),
        ),
    )(h, wr)


def _dispatch_body(h_hbm, d0_hbm, d1_hbm, xs_hbm, idx0_v, idx1_v, rows_v, sem):
    wid = lax.axis_index("s") * NC + lax.axis_index("c")
    base = wid * TPW
    pltpu.sync_copy(d0_hbm.at[pl.ds(base, TPW)], idx0_v)
    pltpu.sync_copy(d1_hbm.at[pl.ds(base, TPW)], idx1_v)
    pltpu.sync_copy(h_hbm.at[pl.ds(base, TPW)], rows_v)
    c0 = pltpu.async_copy(rows_v, xs_hbm.at[idx0_v], sem)
    c0.wait()
    c1 = pltpu.async_copy(rows_v, xs_hbm.at[idx1_v], sem)
    c1.wait()


def _dispatch(h, d0, d1):
    return pl.kernel(
        _dispatch_body,
        out_type=jax.ShapeDtypeStruct((XS, D), jnp.float32),
        mesh=plsc.VectorSubcoreMesh(core_axis_name="c", subcore_axis_name="s"),
        scratch_types=[
            pltpu.VMEM((TPW,), jnp.int32),
            pltpu.VMEM((TPW,), jnp.int32),
            pltpu.VMEM((TPW, D), jnp.float32),
            pltpu.SemaphoreType.DMA,
        ],
    )(h, d0, d1)


def _ffn_body(meta_ref, x_ref, wg_ref, wu_ref, wd_ref, o_ref):
    b = pl.program_id(0)

    @pl.when(b < meta_ref[0])
    def _():
        x = x_ref[...]
        g = jnp.dot(x, wg_ref[0], preferred_element_type=jnp.float32,
                    precision=_PREC)
        u = jnp.dot(x, wu_ref[0], preferred_element_type=jnp.float32,
                    precision=_PREC)
        act = (g * (1.0 / (1.0 + jnp.exp(-g)))) * u
        o_ref[...] = jnp.dot(act, wd_ref[0],
                             preferred_element_type=jnp.float32,
                             precision=_PREC)


def _ffn(meta, xs, wg, wu, wd):
    grid_spec = pltpu.PrefetchScalarGridSpec(
        num_scalar_prefetch=1,
        grid=(NBLK,),
        in_specs=[
            pl.BlockSpec((BLK, D), lambda b, meta: (b, 0)),
            pl.BlockSpec((1, D, F), lambda b, meta: (meta[1 + b], 0, 0)),
            pl.BlockSpec((1, D, F), lambda b, meta: (meta[1 + b], 0, 0)),
            pl.BlockSpec((1, F, D), lambda b, meta: (meta[1 + b], 0, 0)),
        ],
        out_specs=pl.BlockSpec((BLK, D), lambda b, meta: (b, 0)),
    )
    return pl.pallas_call(
        _ffn_body,
        grid_spec=grid_spec,
        out_shape=jax.ShapeDtypeStruct((XS, D), jnp.float32),
        compiler_params=pltpu.CompilerParams(
            dimension_semantics=("arbitrary",)),
    )(meta, xs, wg, wu, wd)


_CCH = 16
_NCH = TPW // _CCH


def _combine_body(ys_hbm, d0_hbm, d1_hbm, w_hbm, out_hbm,
                  idx0_v, idx1_v, w_v, ya_v, yb_v, sem_a, sem_b):
    wid = lax.axis_index("s") * NC + lax.axis_index("c")
    sems = (sem_a, sem_b)

    def issue(c, slot):
        base = wid * TPW + c * _CCH
        pltpu.sync_copy(d0_hbm.at[pl.ds(base, _CCH)], idx0_v.at[slot])
        pltpu.sync_copy(d1_hbm.at[pl.ds(base, _CCH)], idx1_v.at[slot])
        pltpu.sync_copy(w_hbm.at[pl.ds(base, _CCH)], w_v.at[slot])
        ca = pltpu.async_copy(ys_hbm.at[idx0_v.at[slot]], ya_v.at[slot],
                              sems[slot])
        cb = pltpu.async_copy(ys_hbm.at[idx1_v.at[slot]], yb_v.at[slot],
                              sems[slot])
        return ca, cb

    handles = {0: issue(0, 0), 1: issue(1, 1)}
    for c in range(_NCH):
        slot = c % 2
        ca, cb = handles.pop(c)
        ca.wait()
        cb.wait()

        def row_fn(r, _, slot=slot):
            w0v = w_v[slot, r, pl.ds(0, 16)]
            w1v = w_v[slot, r, pl.ds(16, 16)]
            for cc in range(D // 16):
                a = ya_v[slot, r, pl.ds(cc * 16, 16)]
                b = yb_v[slot, r, pl.ds(cc * 16, 16)]
                ya_v[slot, r, pl.ds(cc * 16, 16)] = w0v * a + w1v * b
            return 0

        lax.fori_loop(0, _CCH, row_fn, 0)
        base = wid * TPW + c * _CCH
        pltpu.sync_copy(ya_v.at[slot], out_hbm.at[pl.ds(base, _CCH)])
        if c + 2 < _NCH:
            handles[c + 2] = issue(c + 2, slot)


def _combine(ys, d0, d1, w):
    return pl.kernel(
        _combine_body,
        out_type=jax.ShapeDtypeStruct((T, D), jnp.float32),
        mesh=plsc.VectorSubcoreMesh(core_axis_name="c", subcore_axis_name="s"),
        scratch_types=[
            pltpu.VMEM((2, _CCH), jnp.int32),
            pltpu.VMEM((2, _CCH), jnp.int32),
            pltpu.VMEM((2, _CCH, 32), jnp.float32),
            pltpu.VMEM((2, _CCH, D), jnp.float32),
            pltpu.VMEM((2, _CCH, D), jnp.float32),
            pltpu.SemaphoreType.DMA,
            pltpu.SemaphoreType.DMA,
        ],
    )(ys, d0, d1, w)


def kernel(x, Wr, W_gate, W_up, W_down):
    B, S, _ = x.shape
    h = x.reshape(T, D)
    d0, d1, w, nblk = _router(h, Wr)
    d0 = d0.reshape(T)
    d1 = d1.reshape(T)
    ends = jnp.cumsum(nblk[0])
    total = ends[E - 1]
    b_iota = jnp.arange(NBLK, dtype=jnp.int32)
    blk_e = jnp.sum((b_iota[:, None] >= ends[None, :]).astype(jnp.int32),
                    axis=1)
    e_iota = jnp.arange(E, dtype=jnp.int32)
    e_last = jnp.max(jnp.where(nblk[0] > 0, e_iota, 0))
    blk_e = jnp.minimum(blk_e, e_last)
    meta = jnp.concatenate([total[None], blk_e])
    xs = _dispatch(h, d0, d1)
    ys = _ffn(meta, xs, W_gate, W_up, W_down)
    out = _combine(ys, d0, d1, w)
    return out.reshape(B, S, D)


# speedup vs baseline: 1.3291x; 1.0174x over previous
import jax
import jax.numpy as jnp
from jax import lax
from jax.experimental import pallas as pl
from jax.experimental.pallas import tpu as pltpu
from jax.experimental.pallas import tpu_sc as plsc

D = 1024
F = 2048
E = 8
T = 2048
A = 2 * T
BLK = 128
NBLK = A // BLK + E
XS = NBLK * BLK
NC, NS = 2, 16
NW = NC * NS
TPW = T // NW

_PREC = lax.Precision.DEFAULT


def _router_body(h_ref, wr_ref, dest_ref, w_ref, nblk_ref):
    h = h_ref[...]
    logits = jnp.dot(h, wr_ref[...], preferred_element_type=jnp.float32,
                     precision=_PREC)
    iota = lax.broadcasted_iota(jnp.int32, (T, E), 1)
    m0 = jnp.max(logits, axis=1, keepdims=True)
    e0 = jnp.min(jnp.where(logits == m0, iota, E), axis=1, keepdims=True)
    masked = jnp.where(iota == e0, -jnp.inf, logits)
    m1 = jnp.max(masked, axis=1, keepdims=True)
    e1 = jnp.min(jnp.where(masked == m1, iota, E), axis=1, keepdims=True)
    t = jnp.exp(m1 - m0)
    w0 = 1.0 / (1.0 + t)
    w1 = t * w0

    oh0 = (iota == e0).astype(jnp.int32)
    oh1 = (iota == e1).astype(jnp.int32)
    occ = jnp.concatenate([oh0, oh1], axis=0)
    row = lax.broadcasted_iota(jnp.int32, (A, E), 0)
    s = occ
    d = 1
    while d < A:
        s = s + jnp.where(row >= d, pltpu.roll(s, d, 0), 0)
        d *= 2
    excl = s - occ
    counts = s[A - 1:A, :]
    nblk = (counts + (BLK - 1)) >> 7
    lane = lax.broadcasted_iota(jnp.int32, (1, E), 1)
    c = nblk
    d = 1
    while d < E:
        c = c + jnp.where(lane >= d, pltpu.roll(c, d, 1), 0)
        d *= 2
    pad_start = (c - nblk) << 7

    rank0 = jnp.sum(excl[:T] * oh0, axis=1, keepdims=True)
    rank1 = jnp.sum(excl[T:] * oh1, axis=1, keepdims=True)
    ps0 = jnp.sum(pad_start * oh0, axis=1, keepdims=True)
    ps1 = jnp.sum(pad_start * oh1, axis=1, keepdims=True)
    dest_ref[:, 0:1] = ps0 + rank0
    dest_ref[:, 1:2] = ps1 + rank1
    w_ref[:, 0:16] = jnp.broadcast_to(w0, (T, 16))
    w_ref[:, 16:32] = jnp.broadcast_to(w1, (T, 16))
    nblk_ref[...] = nblk


def _router(h, wr):
    return pl.pallas_call(
        _router_body,
        out_shape=(
            jax.ShapeDtypeStruct((T, 2), jnp.int32),
            jax.ShapeDtypeStruct((T, 32), jnp.float32),
            jax.ShapeDtypeStruct((1, E), jnp.int32),
        ),
    )(h, wr)


def _dispatch_body(h_hbm, d0_hbm, d1_hbm, xs_hbm, idx0_v, idx1_v, rows_v, sem):
    wid = lax.axis_index("s") * NC + lax.axis_index("c")
    base = wid * TPW
    pltpu.sync_copy(d0_hbm.at[pl.ds(base, TPW)], idx0_v)
    pltpu.sync_copy(d1_hbm.at[pl.ds(base, TPW)], idx1_v)
    pltpu.sync_copy(h_hbm.at[pl.ds(base, TPW)], rows_v)
    c0 = pltpu.async_copy(rows_v, xs_hbm.at[idx0_v], sem)
    c0.wait()
    c1 = pltpu.async_copy(rows_v, xs_hbm.at[idx1_v], sem)
    c1.wait()


def _dispatch(h, d0, d1):
    return pl.kernel(
        _dispatch_body,
        out_type=jax.ShapeDtypeStruct((XS, D), jnp.float32),
        mesh=plsc.VectorSubcoreMesh(core_axis_name="c", subcore_axis_name="s"),
        scratch_types=[
            pltpu.VMEM((TPW,), jnp.int32),
            pltpu.VMEM((TPW,), jnp.int32),
            pltpu.VMEM((TPW, D), jnp.float32),
            pltpu.SemaphoreType.DMA,
        ],
    )(h, d0, d1)


def _ffn_body(meta_ref, x_ref, wg_ref, wu_ref, wd_ref, o_ref):
    b = pl.program_id(0)

    @pl.when(b < meta_ref[0])
    def _():
        x = x_ref[...]
        g = jnp.dot(x, wg_ref[0], preferred_element_type=jnp.float32,
                    precision=_PREC)
        u = jnp.dot(x, wu_ref[0], preferred_element_type=jnp.float32,
                    precision=_PREC)
        act = (g * (1.0 / (1.0 + jnp.exp(-g)))) * u
        o_ref[...] = jnp.dot(act, wd_ref[0],
                             preferred_element_type=jnp.float32,
                             precision=_PREC)


def _ffn(meta, xs, wg, wu, wd):
    grid_spec = pltpu.PrefetchScalarGridSpec(
        num_scalar_prefetch=1,
        grid=(NBLK,),
        in_specs=[
            pl.BlockSpec((BLK, D), lambda b, meta: (b, 0)),
            pl.BlockSpec((1, D, F), lambda b, meta: (meta[1 + b], 0, 0)),
            pl.BlockSpec((1, D, F), lambda b, meta: (meta[1 + b], 0, 0)),
            pl.BlockSpec((1, F, D), lambda b, meta: (meta[1 + b], 0, 0)),
        ],
        out_specs=pl.BlockSpec((BLK, D), lambda b, meta: (b, 0)),
    )
    return pl.pallas_call(
        _ffn_body,
        grid_spec=grid_spec,
        out_shape=jax.ShapeDtypeStruct((XS, D), jnp.float32),
        compiler_params=pltpu.CompilerParams(
            dimension_semantics=("arbitrary",)),
    )(meta, xs, wg, wu, wd)


_CCH = 32


def _combine_body(ys_hbm, d0_hbm, d1_hbm, w_hbm, out_hbm,
                  idx0_v, idx1_v, w_v, ya_v, yb_v, sem):
    wid = lax.axis_index("s") * NC + lax.axis_index("c")
    for chunk in range(TPW // _CCH):
        base = wid * TPW + chunk * _CCH
        pltpu.sync_copy(d0_hbm.at[pl.ds(base, _CCH)], idx0_v)
        pltpu.sync_copy(d1_hbm.at[pl.ds(base, _CCH)], idx1_v)
        pltpu.sync_copy(w_hbm.at[pl.ds(base, _CCH)], w_v)
        ca = pltpu.async_copy(ys_hbm.at[idx0_v], ya_v, sem)
        cb = pltpu.async_copy(ys_hbm.at[idx1_v], yb_v, sem)
        ca.wait()
        cb.wait()

        def row_fn(r, _):
            w0v = w_v[r, pl.ds(0, 16)]
            w1v = w_v[r, pl.ds(16, 16)]
            for cc in range(D // 16):
                a = ya_v[r, pl.ds(cc * 16, 16)]
                b = yb_v[r, pl.ds(cc * 16, 16)]
                ya_v[r, pl.ds(cc * 16, 16)] = w0v * a + w1v * b
            return 0

        lax.fori_loop(0, _CCH, row_fn, 0)
        pltpu.sync_copy(ya_v, out_hbm.at[pl.ds(base, _CCH)])


def _combine(ys, d0, d1, w):
    return pl.kernel(
        _combine_body,
        out_type=jax.ShapeDtypeStruct((T, D), jnp.float32),
        mesh=plsc.VectorSubcoreMesh(core_axis_name="c", subcore_axis_name="s"),
        scratch_types=[
            pltpu.VMEM((_CCH,), jnp.int32),
            pltpu.VMEM((_CCH,), jnp.int32),
            pltpu.VMEM((_CCH, 32), jnp.float32),
            pltpu.VMEM((_CCH, D), jnp.float32),
            pltpu.VMEM((_CCH, D), jnp.float32),
            pltpu.SemaphoreType.DMA,
        ],
    )(ys, d0, d1, w)


def kernel(x, Wr, W_gate, W_up, W_down):
    B, S, _ = x.shape
    h = x.reshape(T, D)
    dest, w, nblk = _router(h, Wr)
    d0 = dest[:, 0]
    d1 = dest[:, 1]
    ends = jnp.cumsum(nblk[0])
    total = ends[E - 1]
    b_iota = jnp.arange(NBLK, dtype=jnp.int32)
    blk_e = jnp.sum((b_iota[:, None] >= ends[None, :]).astype(jnp.int32),
                    axis=1)
    e_iota = jnp.arange(E, dtype=jnp.int32)
    e_last = jnp.max(jnp.where(nblk[0] > 0, e_iota, 0))
    blk_e = jnp.minimum(blk_e, e_last)
    meta = jnp.concatenate([total[None], blk_e])
    xs = _dispatch(h, d0, d1)
    ys = _ffn(meta, xs, W_gate, W_up, W_down)
    out = _combine(ys, d0, d1, w)
    return out.reshape(B, S, D)
